# baseline (speedup 1.0000x reference)
import functools

import jax
import jax.numpy as jnp
from jax import lax
from jax.experimental import pallas as pl
from jax.experimental.pallas import tpu as pltpu

E = 8
K = 2
T = 2048
D = 1024
F = 2048

TB = 256
FC = 1024


def _router_body(x_ref, gw_ref, logits_ref, comb_ref):
    x = x_ref[...]
    gw = gw_ref[...]
    logits = lax.dot_general(
        x, gw, (((1,), (1,)), ((), ())),
        preferred_element_type=jnp.float32)
    logits_ref[...] = logits

    m = jnp.max(logits, axis=1, keepdims=True)
    ex = jnp.exp(logits - m)
    probs = ex / jnp.sum(ex, axis=1, keepdims=True)

    iota = lax.broadcasted_iota(jnp.int32, (T, E), 1)
    m1 = jnp.max(probs, axis=1, keepdims=True)
    i1 = jnp.min(jnp.where(probs == m1, iota, E), axis=1, keepdims=True)
    probs2 = jnp.where(iota == i1, -1.0, probs)
    m2 = jnp.max(probs2, axis=1, keepdims=True)
    i2 = jnp.min(jnp.where(probs2 == m2, iota, E), axis=1, keepdims=True)
    s = m1 + m2
    comb = jnp.where(iota == i1, m1 / s, 0.0) + jnp.where(iota == i2, m2 / s, 0.0)
    comb_ref[...] = comb


def _ffn_body(x_ref, w1_ref, w3_ref, w2_ref, comb_ref, out_ref):
    e = pl.program_id(0)
    fc = pl.program_id(1)
    tb = pl.program_id(2)
    rows = pl.ds(tb * TB, TB)

    xb = x_ref[rows, :]
    w1b = w1_ref[0]
    w3b = w3_ref[0]
    w2b = w2_ref[0]

    a = lax.dot_general(xb, w1b, (((1,), (1,)), ((), ())),
                        preferred_element_type=jnp.float32)
    u = lax.dot_general(xb, w3b, (((1,), (1,)), ((), ())),
                        preferred_element_type=jnp.float32)
    g = a * jax.nn.sigmoid(a)
    h = (g * u).astype(jnp.bfloat16)
    y = lax.dot_general(h, w2b, (((1,), (1,)), ((), ())),
                        preferred_element_type=jnp.float32)

    cb = comb_ref[rows, :]
    lane = lax.broadcasted_iota(jnp.int32, (TB, E), 1)
    wcol = jnp.sum(jnp.where(lane == e, cb, 0.0), axis=1, keepdims=True)
    contrib = y * wcol

    @pl.when(jnp.logical_and(e == 0, fc == 0))
    def _():
        out_ref[rows, :] = contrib

    @pl.when(jnp.logical_not(jnp.logical_and(e == 0, fc == 0)))
    def _():
        out_ref[rows, :] = out_ref[rows, :] + contrib


@jax.jit
def _moe(hidden_states, gate_w, w1, w2, w3):
    x = hidden_states.reshape(T, D)

    logits, comb = pl.pallas_call(
        _router_body,
        out_shape=(
            jax.ShapeDtypeStruct((T, E), jnp.float32),
            jax.ShapeDtypeStruct((T, E), jnp.float32),
        ),
    )(x, gate_w)

    xb = x.astype(jnp.bfloat16)
    w1b = w1.astype(jnp.bfloat16)
    w2b = w2.astype(jnp.bfloat16)
    w3b = w3.astype(jnp.bfloat16)

    final = pl.pallas_call(
        _ffn_body,
        grid=(E, F // FC, T // TB),
        in_specs=[
            pl.BlockSpec((T, D), lambda e, fc, tb: (0, 0)),
            pl.BlockSpec((1, FC, D), lambda e, fc, tb: (e, fc, 0)),
            pl.BlockSpec((1, FC, D), lambda e, fc, tb: (e, fc, 0)),
            pl.BlockSpec((1, D, FC), lambda e, fc, tb: (e, 0, fc)),
            pl.BlockSpec((T, E), lambda e, fc, tb: (0, 0)),
        ],
        out_specs=pl.BlockSpec((T, D), lambda e, fc, tb: (0, 0)),
        out_shape=jax.ShapeDtypeStruct((T, D), jnp.float32),
        compiler_params=pltpu.CompilerParams(
            dimension_semantics=("arbitrary", "arbitrary", "arbitrary"),
        ),
    )(xb, w1b, w3b, w2b, comb)

    return final.reshape(1, T, D), logits


def kernel(hidden_states, gate_w, w1, w2, w3):
    return _moe(hidden_states, gate_w, w1, w2, w3)


# speedup vs baseline: 1.2191x; 1.2191x over previous
import functools

import jax
import jax.numpy as jnp
from jax import lax
from jax.experimental import pallas as pl
from jax.experimental.pallas import tpu as pltpu

E = 8
K = 2
T = 2048
D = 1024
F = 2048

TB = 256
FC = 1024


def _router_body(x_ref, gw_ref, logits_ref, comb_ref):
    x = x_ref[...]
    gw = gw_ref[...]
    logits = lax.dot_general(
        x, gw, (((1,), (1,)), ((), ())),
        preferred_element_type=jnp.float32)
    logits_ref[...] = logits

    m = jnp.max(logits, axis=1, keepdims=True)
    ex = jnp.exp(logits - m)
    probs = ex / jnp.sum(ex, axis=1, keepdims=True)

    iota = lax.broadcasted_iota(jnp.int32, (T, E), 1)
    m1 = jnp.max(probs, axis=1, keepdims=True)
    i1 = jnp.min(jnp.where(probs == m1, iota, E), axis=1, keepdims=True)
    probs2 = jnp.where(iota == i1, -1.0, probs)
    m2 = jnp.max(probs2, axis=1, keepdims=True)
    i2 = jnp.min(jnp.where(probs2 == m2, iota, E), axis=1, keepdims=True)
    s = m1 + m2
    comb = jnp.where(iota == i1, m1 / s, 0.0) + jnp.where(iota == i2, m2 / s, 0.0)
    comb_ref[...] = comb


def _ffn_body(x_ref, w1_ref, w3_ref, w2_ref, comb_ref, out_ref):
    e = pl.program_id(0)
    fc = pl.program_id(1)
    tb = pl.program_id(2)
    rows = pl.ds(tb * TB, TB)

    xb = x_ref[rows, :]
    w1b = w1_ref[0]
    w3b = w3_ref[0]
    w2b = w2_ref[0]

    a = lax.dot_general(xb, w1b, (((1,), (1,)), ((), ())),
                        preferred_element_type=jnp.float32)
    u = lax.dot_general(xb, w3b, (((1,), (1,)), ((), ())),
                        preferred_element_type=jnp.float32)
    g = a * jax.nn.sigmoid(a)
    h = g * u
    y = lax.dot_general(h, w2b, (((1,), (1,)), ((), ())),
                        preferred_element_type=jnp.float32)

    cb = comb_ref[rows, :]
    lane = lax.broadcasted_iota(jnp.int32, (TB, E), 1)
    wcol = jnp.sum(jnp.where(lane == e, cb, 0.0), axis=1, keepdims=True)
    contrib = y * wcol

    @pl.when(jnp.logical_and(e == 0, fc == 0))
    def _():
        out_ref[rows, :] = contrib

    @pl.when(jnp.logical_not(jnp.logical_and(e == 0, fc == 0)))
    def _():
        out_ref[rows, :] = out_ref[rows, :] + contrib


@jax.jit
def _moe(hidden_states, gate_w, w1, w2, w3):
    x = hidden_states.reshape(T, D)

    logits, comb = pl.pallas_call(
        _router_body,
        out_shape=(
            jax.ShapeDtypeStruct((T, E), jnp.float32),
            jax.ShapeDtypeStruct((T, E), jnp.float32),
        ),
    )(x, gate_w)

    final = pl.pallas_call(
        _ffn_body,
        grid=(E, F // FC, T // TB),
        in_specs=[
            pl.BlockSpec((T, D), lambda e, fc, tb: (0, 0)),
            pl.BlockSpec((1, FC, D), lambda e, fc, tb: (e, fc, 0)),
            pl.BlockSpec((1, FC, D), lambda e, fc, tb: (e, fc, 0)),
            pl.BlockSpec((1, D, FC), lambda e, fc, tb: (e, 0, fc)),
            pl.BlockSpec((T, E), lambda e, fc, tb: (0, 0)),
        ],
        out_specs=pl.BlockSpec((T, D), lambda e, fc, tb: (0, 0)),
        out_shape=jax.ShapeDtypeStruct((T, D), jnp.float32),
        compiler_params=pltpu.CompilerParams(
            dimension_semantics=("arbitrary", "arbitrary", "arbitrary"),
        ),
    )(x, w1, w3, w2, comb)

    return final.reshape(1, T, D), logits


def kernel(hidden_states, gate_w, w1, w2, w3):
    return _moe(hidden_states, gate_w, w1, w2, w3)


# speedup vs baseline: 1.4360x; 1.1779x over previous
import jax
import jax.numpy as jnp
from jax import lax
from jax.experimental import pallas as pl
from jax.experimental.pallas import tpu as pltpu
from jax.experimental.pallas import tpu_sc as plsc

E = 8
K = 2
T = 2048
D = 1024
F = 2048
A = T * K

BLK = 256
MAXB = 24
PADN = MAXB * BLK
FC = 1024
CB = 512


def _router_body(x_ref, gw_ref, logits_ref, pos0_ref, pos1_ref,
                 ww0_ref, ww1_ref, bemeta_ref):
    x = x_ref[...]
    gw = gw_ref[...]
    logits = lax.dot_general(x, gw, (((1,), (1,)), ((), ())),
                             preferred_element_type=jnp.float32)
    logits_ref[...] = logits

    m = jnp.max(logits, axis=1, keepdims=True)
    ex = jnp.exp(logits - m)
    probs = ex / jnp.sum(ex, axis=1, keepdims=True)

    iota = lax.broadcasted_iota(jnp.int32, (T, E), 1)
    m1 = jnp.max(probs, axis=1, keepdims=True)
    i1 = jnp.min(jnp.where(probs == m1, iota, E), axis=1, keepdims=True)
    probs2 = jnp.where(iota == i1, -1.0, probs)
    m2 = jnp.max(probs2, axis=1, keepdims=True)
    i2 = jnp.min(jnp.where(probs2 == m2, iota, E), axis=1, keepdims=True)
    s = m1 + m2
    ww0_ref[...] = jnp.broadcast_to(m1 / s, (T, 16))
    ww1_ref[...] = jnp.broadcast_to(m2 / s, (T, 16))

    o0 = jnp.where(iota == i1, 1.0, 0.0)
    o1 = jnp.where(iota == i2, 1.0, 0.0)

    cnt = (jnp.sum(o0, axis=0, keepdims=True)
           + jnp.sum(o1, axis=0, keepdims=True))
    cnt_i = cnt.astype(jnp.int32)
    nb = (cnt_i + (BLK - 1)) // BLK
    nbf = nb.astype(jnp.float32)

    ei = lax.broadcasted_iota(jnp.int32, (E, E), 0)
    ej = lax.broadcasted_iota(jnp.int32, (E, E), 1)
    triu_strict = jnp.where(ei < ej, 1.0, 0.0)
    start = lax.dot_general(nbf, triu_strict, (((1,), (0,)), ((), ())),
                            preferred_element_type=jnp.float32) * BLK

    bi = lax.broadcasted_iota(jnp.int32, (CB, CB), 0)
    bj = lax.broadcasted_iota(jnp.int32, (CB, CB), 1)
    tril_strict = jnp.where(bi > bj, 1.0, 0.0)

    carry = jnp.zeros((1, E), jnp.float32)
    for oh, pref in ((o0, pos0_ref), (o1, pos1_ref)):
        for b in range(T // CB):
            ob = oh[b * CB:(b + 1) * CB, :]
            rb = lax.dot_general(tril_strict, ob, (((1,), (0,)), ((), ())),
                                 preferred_element_type=jnp.float32) + carry
            carry = carry + jnp.sum(ob, axis=0, keepdims=True)
            p = jnp.sum(ob * (start + rb), axis=1, keepdims=True)
            pref[b * CB:(b + 1) * CB, :] = p.astype(jnp.int32)

    tril_incl = jnp.where(ei <= ej, 1.0, 0.0)
    nbs = lax.dot_general(nbf, tril_incl, (((1,), (0,)), ((), ())),
                          preferred_element_type=jnp.float32).astype(jnp.int32)
    nblk = nbs[0:1, E - 1:E]
    lane32 = lax.broadcasted_iota(jnp.int32, (1, 32), 1)
    bev = jnp.zeros((1, 32), jnp.int32)
    last_e = jnp.zeros((1, 1), jnp.int32)
    for e in range(E):
        nbs_e = nbs[0:1, e:e + 1]
        bev = bev + jnp.where(nbs_e <= lane32, 1, 0)
        last_e = last_e + jnp.where(nbs_e < nblk, 1, 0)
    val = jnp.where(lane32 < nblk, bev, last_e)
    val = jnp.where(lane32 == MAXB, nblk, val)
    bemeta_ref[...] = val


GCH = 32


def _gather_body(x_hbm, pos3_hbm, xs_hbm, idx_v, rows_v, sem):
    c = lax.axis_index("c")
    s = lax.axis_index("s")
    u = s * 2 + c
    tok0 = (u % 16) * 128

    pltpu.sync_copy(pos3_hbm.at[u], idx_v)
    for ch in range(128 // GCH):
        pltpu.sync_copy(x_hbm.at[pl.ds(tok0 + ch * GCH, GCH)], rows_v)
        pltpu.async_copy(rows_v, xs_hbm.at[idx_v.at[ch]], sem).wait()


def _ffn_body(be_sm, xs_ref, w1_ref, w3_ref, w2_ref, ys_ref):
    b = pl.program_id(0)
    fc = pl.program_id(1)
    nblk = be_sm[MAXB]

    @pl.when(b < nblk)
    def _():
        xb = xs_ref[...]
        a = lax.dot_general(xb, w1_ref[0], (((1,), (1,)), ((), ())),
                            preferred_element_type=jnp.float32)
        u = lax.dot_general(xb, w3_ref[0], (((1,), (1,)), ((), ())),
                            preferred_element_type=jnp.float32)
        h = (a * jax.nn.sigmoid(a)) * u
        y = lax.dot_general(h, w2_ref[0], (((1,), (1,)), ((), ())),
                            preferred_element_type=jnp.float32)

        @pl.when(fc == 0)
        def _():
            ys_ref[...] = y

        @pl.when(fc != 0)
        def _():
            ys_ref[...] = ys_ref[...] + y


CCH = 16


def _combine_body(ys_hbm, pos2_hbm, ww0_hbm, ww1_hbm, out_hbm,
                  idx0_v, idx1_v, w0_v, w1_v, r0_v, r1_v, o_v, sem):
    c = lax.axis_index("c")
    s = lax.axis_index("s")
    u = s * 2 + c
    tb = u * 64

    pltpu.sync_copy(pos2_hbm.at[0, pl.ds(tb, 64)], idx0_v)
    pltpu.sync_copy(pos2_hbm.at[1, pl.ds(tb, 64)], idx1_v)
    pltpu.sync_copy(ww0_hbm.at[pl.ds(tb, 64)], w0_v)
    pltpu.sync_copy(ww1_hbm.at[pl.ds(tb, 64)], w1_v)

    for ch in range(64 // CCH):
        pltpu.async_copy(ys_hbm.at[idx0_v.at[pl.ds(ch * CCH, CCH)]],
                         r0_v, sem).wait()
        pltpu.async_copy(ys_hbm.at[idx1_v.at[pl.ds(ch * CCH, CCH)]],
                         r1_v, sem).wait()

        for t in range(CCH):
            s0 = w0_v[ch * CCH + t]
            s1 = w1_v[ch * CCH + t]

            def vec_step(v, _, t=t, s0=s0, s1=s1):
                sl = pl.ds(v * 16, 16)
                o_v[t, sl] = r0_v[t, sl] * s0 + r1_v[t, sl] * s1
                return 0

            lax.fori_loop(0, D // 16, vec_step, 0)

        pltpu.sync_copy(o_v, out_hbm.at[pl.ds(tb + ch * CCH, CCH)])


_MESH = plsc.VectorSubcoreMesh(core_axis_name="c", subcore_axis_name="s")


@jax.jit
def _moe(hidden_states, gate_w, w1, w2, w3):
    x = hidden_states.reshape(T, D)

    logits, pos0, pos1, ww0, ww1, bemeta = pl.pallas_call(
        _router_body,
        out_shape=(
            jax.ShapeDtypeStruct((T, E), jnp.float32),
            jax.ShapeDtypeStruct((T, 1), jnp.int32),
            jax.ShapeDtypeStruct((T, 1), jnp.int32),
            jax.ShapeDtypeStruct((T, 16), jnp.float32),
            jax.ShapeDtypeStruct((T, 16), jnp.float32),
            jax.ShapeDtypeStruct((1, 32), jnp.int32),
        ),
    )(x, gate_w)

    pos = jnp.concatenate([pos0.reshape(1, T), pos1.reshape(1, T)], axis=0)

    pos3 = pos.reshape(32, 128 // GCH, GCH)
    xs = pl.kernel(
        _gather_body,
        mesh=_MESH,
        out_type=[jax.ShapeDtypeStruct((PADN, D), jnp.float32)],
        scratch_types=[
            pltpu.VMEM((128 // GCH, GCH), jnp.int32),
            pltpu.VMEM((GCH, D), jnp.float32),
            pltpu.SemaphoreType.DMA,
        ],
    )(x, pos3)[0]

    ys = pl.pallas_call(
        _ffn_body,
        grid_spec=pltpu.PrefetchScalarGridSpec(
            num_scalar_prefetch=1,
            grid=(MAXB, F // FC),
            in_specs=[
                pl.BlockSpec((BLK, D), lambda b, fc, s: (b, 0)),
                pl.BlockSpec((1, FC, D), lambda b, fc, s: (s[b], fc, 0)),
                pl.BlockSpec((1, FC, D), lambda b, fc, s: (s[b], fc, 0)),
                pl.BlockSpec((1, D, FC), lambda b, fc, s: (s[b], 0, fc)),
            ],
            out_specs=pl.BlockSpec((BLK, D), lambda b, fc, s: (b, 0)),
        ),
        out_shape=jax.ShapeDtypeStruct((PADN, D), jnp.float32),
        compiler_params=pltpu.CompilerParams(
            dimension_semantics=("arbitrary", "arbitrary"),
        ),
    )(bemeta.reshape(32), xs, w1, w3, w2)

    final = pl.kernel(
        _combine_body,
        mesh=_MESH,
        out_type=[jax.ShapeDtypeStruct((T, D), jnp.float32)],
        scratch_types=[
            pltpu.VMEM((64,), jnp.int32),
            pltpu.VMEM((64,), jnp.int32),
            pltpu.VMEM((64, 16), jnp.float32),
            pltpu.VMEM((64, 16), jnp.float32),
            pltpu.VMEM((CCH, D), jnp.float32),
            pltpu.VMEM((CCH, D), jnp.float32),
            pltpu.VMEM((CCH, D), jnp.float32),
            pltpu.SemaphoreType.DMA,
        ],
    )(ys, pos, ww0, ww1)[0]

    return final.reshape(1, T, D), logits


def kernel(hidden_states, gate_w, w1, w2, w3):
    return _moe(hidden_states, gate_w, w1, w2, w3)


# speedup vs baseline: 1.9897x; 1.3856x over previous
import jax
import jax.numpy as jnp
from jax import lax
from jax.experimental import pallas as pl
from jax.experimental.pallas import tpu as pltpu
from jax.experimental.pallas import tpu_sc as plsc

E = 8
K = 2
T = 2048
D = 1024
F = 2048
A = T * K

BLK = 256
MAXB = 24
PADN = MAXB * BLK
CB = 512


def _router_body(x_ref, gw_ref, logits_ref, pos0_ref, pos1_ref,
                 ww0_ref, ww1_ref, bemeta_ref):
    x = x_ref[...]
    gw = gw_ref[...]
    logits = lax.dot_general(x, gw, (((1,), (1,)), ((), ())),
                             preferred_element_type=jnp.float32)
    logits_ref[...] = logits

    m = jnp.max(logits, axis=1, keepdims=True)
    ex = jnp.exp(logits - m)
    probs = ex / jnp.sum(ex, axis=1, keepdims=True)

    iota = lax.broadcasted_iota(jnp.int32, (T, E), 1)
    m1 = jnp.max(probs, axis=1, keepdims=True)
    i1 = jnp.min(jnp.where(probs == m1, iota, E), axis=1, keepdims=True)
    probs2 = jnp.where(iota == i1, -1.0, probs)
    m2 = jnp.max(probs2, axis=1, keepdims=True)
    i2 = jnp.min(jnp.where(probs2 == m2, iota, E), axis=1, keepdims=True)
    s = m1 + m2
    ww0_ref[...] = jnp.broadcast_to(m1 / s, (T, 16))
    ww1_ref[...] = jnp.broadcast_to(m2 / s, (T, 16))

    o0 = jnp.where(iota == i1, 1.0, 0.0)
    o1 = jnp.where(iota == i2, 1.0, 0.0)

    cnt = (jnp.sum(o0, axis=0, keepdims=True)
           + jnp.sum(o1, axis=0, keepdims=True))
    cnt_i = cnt.astype(jnp.int32)
    nb = (cnt_i + (BLK - 1)) // BLK
    nbf = nb.astype(jnp.float32)

    ei = lax.broadcasted_iota(jnp.int32, (E, E), 0)
    ej = lax.broadcasted_iota(jnp.int32, (E, E), 1)
    triu_strict = jnp.where(ei < ej, 1.0, 0.0)
    start = lax.dot_general(nbf, triu_strict, (((1,), (0,)), ((), ())),
                            preferred_element_type=jnp.float32) * BLK

    bi = lax.broadcasted_iota(jnp.int32, (CB, CB), 0)
    bj = lax.broadcasted_iota(jnp.int32, (CB, CB), 1)
    tril_strict = jnp.where(bi > bj, 1.0, 0.0)

    carry = jnp.zeros((1, E), jnp.float32)
    for oh, pref in ((o0, pos0_ref), (o1, pos1_ref)):
        for b in range(T // CB):
            ob = oh[b * CB:(b + 1) * CB, :]
            rb = lax.dot_general(tril_strict, ob, (((1,), (0,)), ((), ())),
                                 preferred_element_type=jnp.float32) + carry
            carry = carry + jnp.sum(ob, axis=0, keepdims=True)
            p = jnp.sum(ob * (start + rb), axis=1, keepdims=True)
            pref[b * CB:(b + 1) * CB, :] = p.astype(jnp.int32)

    tril_incl = jnp.where(ei <= ej, 1.0, 0.0)
    nbs = lax.dot_general(nbf, tril_incl, (((1,), (0,)), ((), ())),
                          preferred_element_type=jnp.float32).astype(jnp.int32)
    nblk = nbs[0:1, E - 1:E]
    lane32 = lax.broadcasted_iota(jnp.int32, (1, 32), 1)
    bev = jnp.zeros((1, 32), jnp.int32)
    last_e = jnp.zeros((1, 1), jnp.int32)
    for e in range(E):
        nbs_e = nbs[0:1, e:e + 1]
        bev = bev + jnp.where(nbs_e <= lane32, 1, 0)
        last_e = last_e + jnp.where(nbs_e < nblk, 1, 0)
    val = jnp.where(lane32 < nblk, bev, last_e)
    val = jnp.where(lane32 == MAXB, nblk, val)
    bemeta_ref[...] = val


GCH = 32


def _gather_body(x_hbm, pos3_hbm, xs_hbm, idx_v, rows_v, sem):
    c = lax.axis_index("c")
    s = lax.axis_index("s")
    u = s * 2 + c
    tok0 = (u % 16) * 128

    pltpu.sync_copy(pos3_hbm.at[u], idx_v)
    for ch in range(128 // GCH):
        pltpu.sync_copy(x_hbm.at[pl.ds(tok0 + ch * GCH, GCH)], rows_v)
        pltpu.async_copy(rows_v, xs_hbm.at[idx_v.at[ch]], sem).wait()


def _ffn_body(be_sm, xs_ref, w1_ref, w3_ref, w2_ref, ys_ref):
    b = pl.program_id(0)
    nblk = be_sm[MAXB]

    @pl.when(b < nblk)
    def _():
        xb = xs_ref[...]
        a = lax.dot_general(xb, w1_ref[0], (((1,), (1,)), ((), ())),
                            preferred_element_type=jnp.float32)
        u = lax.dot_general(xb, w3_ref[0], (((1,), (1,)), ((), ())),
                            preferred_element_type=jnp.float32)
        h = (a * jax.nn.sigmoid(a)) * u
        y = lax.dot_general(h, w2_ref[0], (((1,), (1,)), ((), ())),
                            preferred_element_type=jnp.float32)
        ys_ref[...] = y


CCH = 16
NCH = 64 // CCH


def _combine_body(ys_hbm, pos0_hbm, pos1_hbm, ww0_hbm, ww1_hbm, out_hbm,
                  idx0_v, idx1_v, w0_v, w1_v, r0_v, r1_v, o_v, sem):
    c = lax.axis_index("c")
    s = lax.axis_index("s")
    u = s * 2 + c
    tb = u * 64

    pltpu.sync_copy(pos0_hbm.at[pl.ds(tb, 64)], idx0_v)
    pltpu.sync_copy(pos1_hbm.at[pl.ds(tb, 64)], idx1_v)
    pltpu.sync_copy(ww0_hbm.at[pl.ds(tb, 64)], w0_v)
    pltpu.sync_copy(ww1_hbm.at[pl.ds(tb, 64)], w1_v)

    def issue(ch, buf):
        a = pltpu.async_copy(ys_hbm.at[idx0_v.at[pl.ds(ch * CCH, CCH)]],
                             r0_v.at[buf], sem)
        b = pltpu.async_copy(ys_hbm.at[idx1_v.at[pl.ds(ch * CCH, CCH)]],
                             r1_v.at[buf], sem)
        return a, b

    cps = issue(0, 0)
    for ch in range(NCH):
        cps[0].wait()
        cps[1].wait()
        buf = ch % 2
        if ch + 1 < NCH:
            cps = issue(ch + 1, 1 - buf)

        for t in range(CCH):
            s0 = w0_v[ch * CCH + t]
            s1 = w1_v[ch * CCH + t]

            def vec_step(v, _, t=t, s0=s0, s1=s1, buf=buf):
                sl = pl.ds(v * 16, 16)
                o_v[t, sl] = (r0_v[buf, t, sl] * s0
                              + r1_v[buf, t, sl] * s1)
                return 0

            lax.fori_loop(0, D // 16, vec_step, 0)

        pltpu.sync_copy(o_v, out_hbm.at[pl.ds(tb + ch * CCH, CCH)])


_MESH = plsc.VectorSubcoreMesh(core_axis_name="c", subcore_axis_name="s")


@jax.jit
def _moe(hidden_states, gate_w, w1, w2, w3):
    x = hidden_states.reshape(T, D)

    logits, pos0, pos1, ww0, ww1, bemeta = pl.pallas_call(
        _router_body,
        out_shape=(
            jax.ShapeDtypeStruct((T, E), jnp.float32),
            jax.ShapeDtypeStruct((T, 1), jnp.int32),
            jax.ShapeDtypeStruct((T, 1), jnp.int32),
            jax.ShapeDtypeStruct((T, 16), jnp.float32),
            jax.ShapeDtypeStruct((T, 16), jnp.float32),
            jax.ShapeDtypeStruct((1, 32), jnp.int32),
        ),
    )(x, gate_w)

    pos3 = jnp.concatenate([pos0.reshape(1, T), pos1.reshape(1, T)],
                           axis=0).reshape(32, 128 // GCH, GCH)
    xs = pl.kernel(
        _gather_body,
        mesh=_MESH,
        out_type=[jax.ShapeDtypeStruct((PADN, D), jnp.float32)],
        scratch_types=[
            pltpu.VMEM((128 // GCH, GCH), jnp.int32),
            pltpu.VMEM((GCH, D), jnp.float32),
            pltpu.SemaphoreType.DMA,
        ],
    )(x, pos3)[0]

    ys = pl.pallas_call(
        _ffn_body,
        grid_spec=pltpu.PrefetchScalarGridSpec(
            num_scalar_prefetch=1,
            grid=(MAXB,),
            in_specs=[
                pl.BlockSpec((BLK, D), lambda b, s: (b, 0)),
                pl.BlockSpec((1, F, D), lambda b, s: (s[b], 0, 0)),
                pl.BlockSpec((1, F, D), lambda b, s: (s[b], 0, 0)),
                pl.BlockSpec((1, D, F), lambda b, s: (s[b], 0, 0)),
            ],
            out_specs=pl.BlockSpec((BLK, D), lambda b, s: (b, 0)),
        ),
        out_shape=jax.ShapeDtypeStruct((PADN, D), jnp.float32),
        compiler_params=pltpu.CompilerParams(
            dimension_semantics=("arbitrary",),
        ),
    )(bemeta.reshape(32), xs, w1, w3, w2)

    final = pl.kernel(
        _combine_body,
        mesh=_MESH,
        out_type=[jax.ShapeDtypeStruct((T, D), jnp.float32)],
        scratch_types=[
            pltpu.VMEM((64,), jnp.int32),
            pltpu.VMEM((64,), jnp.int32),
            pltpu.VMEM((64, 16), jnp.float32),
            pltpu.VMEM((64, 16), jnp.float32),
            pltpu.VMEM((2, CCH, D), jnp.float32),
            pltpu.VMEM((2, CCH, D), jnp.float32),
            pltpu.VMEM((CCH, D), jnp.float32),
            pltpu.SemaphoreType.DMA,
        ],
    )(ys, pos0.reshape(T), pos1.reshape(T), ww0, ww1)[0]

    return final.reshape(1, T, D), logits


def kernel(hidden_states, gate_w, w1, w2, w3):
    return _moe(hidden_states, gate_w, w1, w2, w3)


# speedup vs baseline: 2.0480x; 1.0293x over previous
import jax
import jax.numpy as jnp
from jax import lax
from jax.experimental import pallas as pl
from jax.experimental.pallas import tpu as pltpu
from jax.experimental.pallas import tpu_sc as plsc

E = 8
K = 2
T = 2048
D = 1024
F = 2048
A = T * K

BLK = 256
MAXB = 24
PADN = MAXB * BLK
CB = 512


def _router_body(x_ref, gw_ref, logits_ref, pos0_ref, pos1_ref,
                 ww0_ref, ww1_ref, bemeta_ref):
    x = x_ref[...]
    gw = gw_ref[...]
    logits = lax.dot_general(x, gw, (((1,), (1,)), ((), ())),
                             preferred_element_type=jnp.float32)
    logits_ref[...] = logits

    m = jnp.max(logits, axis=1, keepdims=True)
    ex = jnp.exp(logits - m)
    probs = ex / jnp.sum(ex, axis=1, keepdims=True)

    iota = lax.broadcasted_iota(jnp.int32, (T, E), 1)
    m1 = jnp.max(probs, axis=1, keepdims=True)
    i1 = jnp.min(jnp.where(probs == m1, iota, E), axis=1, keepdims=True)
    probs2 = jnp.where(iota == i1, -1.0, probs)
    m2 = jnp.max(probs2, axis=1, keepdims=True)
    i2 = jnp.min(jnp.where(probs2 == m2, iota, E), axis=1, keepdims=True)
    s = m1 + m2
    ww0_ref[...] = jnp.broadcast_to(m1 / s, (T, 16))
    ww1_ref[...] = jnp.broadcast_to(m2 / s, (T, 16))

    o0 = jnp.where(iota == i1, 1.0, 0.0)
    o1 = jnp.where(iota == i2, 1.0, 0.0)

    cnt = (jnp.sum(o0, axis=0, keepdims=True)
           + jnp.sum(o1, axis=0, keepdims=True))
    cnt_i = cnt.astype(jnp.int32)
    nb = (cnt_i + (BLK - 1)) // BLK
    nbf = nb.astype(jnp.float32)

    ei = lax.broadcasted_iota(jnp.int32, (E, E), 0)
    ej = lax.broadcasted_iota(jnp.int32, (E, E), 1)
    triu_strict = jnp.where(ei < ej, 1.0, 0.0)
    start = lax.dot_general(nbf, triu_strict, (((1,), (0,)), ((), ())),
                            preferred_element_type=jnp.float32) * BLK

    bi = lax.broadcasted_iota(jnp.int32, (CB, CB), 0)
    bj = lax.broadcasted_iota(jnp.int32, (CB, CB), 1)
    tril_strict = jnp.where(bi > bj, 1.0, 0.0)

    carry = jnp.zeros((1, E), jnp.float32)
    for oh, pref in ((o0, pos0_ref), (o1, pos1_ref)):
        for b in range(T // CB):
            ob = oh[b * CB:(b + 1) * CB, :]
            rb = lax.dot_general(tril_strict, ob, (((1,), (0,)), ((), ())),
                                 preferred_element_type=jnp.float32) + carry
            carry = carry + jnp.sum(ob, axis=0, keepdims=True)
            p = jnp.sum(ob * (start + rb), axis=1, keepdims=True)
            pref[b * CB:(b + 1) * CB, :] = p.astype(jnp.int32)

    tril_incl = jnp.where(ei <= ej, 1.0, 0.0)
    nbs = lax.dot_general(nbf, tril_incl, (((1,), (0,)), ((), ())),
                          preferred_element_type=jnp.float32).astype(jnp.int32)
    nblk = nbs[0:1, E - 1:E]
    lane32 = lax.broadcasted_iota(jnp.int32, (1, 32), 1)
    bev = jnp.zeros((1, 32), jnp.int32)
    last_e = jnp.zeros((1, 1), jnp.int32)
    for e in range(E):
        nbs_e = nbs[0:1, e:e + 1]
        bev = bev + jnp.where(nbs_e <= lane32, 1, 0)
        last_e = last_e + jnp.where(nbs_e < nblk, 1, 0)
    val = jnp.where(lane32 < nblk, bev, last_e)
    val = jnp.where(lane32 == MAXB, nblk, val)
    bemeta_ref[...] = val


GCH = 32


def _gather_body(x_hbm, pos3_hbm, xs_hbm, idx_v, rows_v, sem):
    c = lax.axis_index("c")
    s = lax.axis_index("s")
    u = s * 2 + c
    tok0 = (u % 16) * 128

    pltpu.sync_copy(pos3_hbm.at[u], idx_v)
    for ch in range(128 // GCH):
        pltpu.sync_copy(x_hbm.at[pl.ds(tok0 + ch * GCH, GCH)], rows_v)
        pltpu.async_copy(rows_v, xs_hbm.at[idx_v.at[ch]], sem).wait()


def _ffn_body(be_sm, xs_ref, w1_ref, w3_ref, w2_ref, ys_ref):
    b = pl.program_id(0)
    nblk = be_sm[MAXB]

    @pl.when(b < nblk)
    def _():
        xb = xs_ref[...]
        a = lax.dot_general(xb, w1_ref[0], (((1,), (1,)), ((), ())),
                            preferred_element_type=jnp.float32)
        u = lax.dot_general(xb, w3_ref[0], (((1,), (1,)), ((), ())),
                            preferred_element_type=jnp.float32)
        h = (a * jax.nn.sigmoid(a)) * u
        y = lax.dot_general(h, w2_ref[0], (((1,), (1,)), ((), ())),
                            preferred_element_type=jnp.float32)
        ys_ref[...] = y


CCH = 16
NCH = 64 // CCH


def _combine_body(ys_hbm, pos0_hbm, pos1_hbm, ww0_hbm, ww1_hbm, out_hbm,
                  idx0_v, idx1_v, w0_v, w1_v, r0_v, r1_v, o_v, sem):
    c = lax.axis_index("c")
    s = lax.axis_index("s")
    u = s * 2 + c
    tb = u * 64

    pltpu.sync_copy(pos0_hbm.at[pl.ds(tb, 64)], idx0_v)
    pltpu.sync_copy(pos1_hbm.at[pl.ds(tb, 64)], idx1_v)
    pltpu.sync_copy(ww0_hbm.at[pl.ds(tb, 64)], w0_v)
    pltpu.sync_copy(ww1_hbm.at[pl.ds(tb, 64)], w1_v)

    def issue(ch, buf):
        a = pltpu.async_copy(ys_hbm.at[idx0_v.at[pl.ds(ch * CCH, CCH)]],
                             r0_v.at[buf], sem)
        b = pltpu.async_copy(ys_hbm.at[idx1_v.at[pl.ds(ch * CCH, CCH)]],
                             r1_v.at[buf], sem)
        return a, b

    cps = issue(0, 0)
    for ch in range(NCH):
        cps[0].wait()
        cps[1].wait()
        buf = ch % 2
        if ch + 1 < NCH:
            cps = issue(ch + 1, 1 - buf)

        def tok_step(t, _, ch=ch, buf=buf):
            s0 = w0_v[ch * CCH + t]
            s1 = w1_v[ch * CCH + t]
            for v in range(D // 16):
                sl = pl.ds(v * 16, 16)
                o_v[t, sl] = (r0_v[buf, t, sl] * s0
                              + r1_v[buf, t, sl] * s1)
            return 0

        lax.fori_loop(0, CCH, tok_step, 0)
        pltpu.sync_copy(o_v, out_hbm.at[pl.ds(tb + ch * CCH, CCH)])


_MESH = plsc.VectorSubcoreMesh(core_axis_name="c", subcore_axis_name="s")


@jax.jit
def _moe(hidden_states, gate_w, w1, w2, w3):
    x = hidden_states.reshape(T, D)

    logits, pos0, pos1, ww0, ww1, bemeta = pl.pallas_call(
        _router_body,
        out_shape=(
            jax.ShapeDtypeStruct((T, E), jnp.float32),
            jax.ShapeDtypeStruct((T, 1), jnp.int32),
            jax.ShapeDtypeStruct((T, 1), jnp.int32),
            jax.ShapeDtypeStruct((T, 16), jnp.float32),
            jax.ShapeDtypeStruct((T, 16), jnp.float32),
            jax.ShapeDtypeStruct((1, 32), jnp.int32),
        ),
    )(x, gate_w)

    pos3 = jnp.concatenate([pos0.reshape(1, T), pos1.reshape(1, T)],
                           axis=0).reshape(32, 128 // GCH, GCH)
    xs = pl.kernel(
        _gather_body,
        mesh=_MESH,
        out_type=[jax.ShapeDtypeStruct((PADN, D), jnp.float32)],
        scratch_types=[
            pltpu.VMEM((128 // GCH, GCH), jnp.int32),
            pltpu.VMEM((GCH, D), jnp.float32),
            pltpu.SemaphoreType.DMA,
        ],
    )(x, pos3)[0]

    ys = pl.pallas_call(
        _ffn_body,
        grid_spec=pltpu.PrefetchScalarGridSpec(
            num_scalar_prefetch=1,
            grid=(MAXB,),
            in_specs=[
                pl.BlockSpec((BLK, D), lambda b, s: (b, 0)),
                pl.BlockSpec((1, F, D), lambda b, s: (s[b], 0, 0)),
                pl.BlockSpec((1, F, D), lambda b, s: (s[b], 0, 0)),
                pl.BlockSpec((1, D, F), lambda b, s: (s[b], 0, 0)),
            ],
            out_specs=pl.BlockSpec((BLK, D), lambda b, s: (b, 0)),
        ),
        out_shape=jax.ShapeDtypeStruct((PADN, D), jnp.float32),
        compiler_params=pltpu.CompilerParams(
            dimension_semantics=("arbitrary",),
        ),
    )(bemeta.reshape(32), xs, w1, w3, w2)

    final = pl.kernel(
        _combine_body,
        mesh=_MESH,
        out_type=[jax.ShapeDtypeStruct((T, D), jnp.float32)],
        scratch_types=[
            pltpu.VMEM((64,), jnp.int32),
            pltpu.VMEM((64,), jnp.int32),
            pltpu.VMEM((64, 16), jnp.float32),
            pltpu.VMEM((64, 16), jnp.float32),
            pltpu.VMEM((2, CCH, D), jnp.float32),
            pltpu.VMEM((2, CCH, D), jnp.float32),
            pltpu.VMEM((CCH, D), jnp.float32),
            pltpu.SemaphoreType.DMA,
        ],
    )(ys, pos0.reshape(T), pos1.reshape(T), ww0, ww1)[0]

    return final.reshape(1, T, D), logits


def kernel(hidden_states, gate_w, w1, w2, w3):
    return _moe(hidden_states, gate_w, w1, w2, w3)


# speedup vs baseline: 2.0513x; 1.0016x over previous
import jax
import jax.numpy as jnp
from jax import lax
from jax.experimental import pallas as pl
from jax.experimental.pallas import tpu as pltpu
from jax.experimental.pallas import tpu_sc as plsc

E = 8
K = 2
T = 2048
D = 1024
F = 2048
A = T * K

BLK = 256
MAXB = 24
PADN = MAXB * BLK
CB = 512


def _router_body(x_ref, gw_ref, logits_ref, pos0_ref, pos1_ref,
                 ww0_ref, ww1_ref, bemeta_ref):
    x = x_ref[...]
    gw = gw_ref[...]
    logits = lax.dot_general(x, gw, (((1,), (1,)), ((), ())),
                             preferred_element_type=jnp.float32)
    logits_ref[...] = logits

    m = jnp.max(logits, axis=1, keepdims=True)
    ex = jnp.exp(logits - m)
    probs = ex / jnp.sum(ex, axis=1, keepdims=True)

    iota = lax.broadcasted_iota(jnp.int32, (T, E), 1)
    m1 = jnp.max(probs, axis=1, keepdims=True)
    i1 = jnp.min(jnp.where(probs == m1, iota, E), axis=1, keepdims=True)
    probs2 = jnp.where(iota == i1, -1.0, probs)
    m2 = jnp.max(probs2, axis=1, keepdims=True)
    i2 = jnp.min(jnp.where(probs2 == m2, iota, E), axis=1, keepdims=True)
    s = m1 + m2
    ww0_ref[...] = jnp.broadcast_to(m1 / s, (T, 16))
    ww1_ref[...] = jnp.broadcast_to(m2 / s, (T, 16))

    o0 = jnp.where(iota == i1, 1.0, 0.0)
    o1 = jnp.where(iota == i2, 1.0, 0.0)

    cnt = (jnp.sum(o0, axis=0, keepdims=True)
           + jnp.sum(o1, axis=0, keepdims=True))
    cnt_i = cnt.astype(jnp.int32)
    nb = (cnt_i + (BLK - 1)) // BLK
    nbf = nb.astype(jnp.float32)

    ei = lax.broadcasted_iota(jnp.int32, (E, E), 0)
    ej = lax.broadcasted_iota(jnp.int32, (E, E), 1)
    triu_strict = jnp.where(ei < ej, 1.0, 0.0)
    start = lax.dot_general(nbf, triu_strict, (((1,), (0,)), ((), ())),
                            preferred_element_type=jnp.float32) * BLK

    bi = lax.broadcasted_iota(jnp.int32, (CB, CB), 0)
    bj = lax.broadcasted_iota(jnp.int32, (CB, CB), 1)
    tril_strict = jnp.where(bi > bj, 1.0, 0.0)

    carry = jnp.zeros((1, E), jnp.float32)
    for oh, pref in ((o0, pos0_ref), (o1, pos1_ref)):
        for b in range(T // CB):
            ob = oh[b * CB:(b + 1) * CB, :]
            rb = lax.dot_general(tril_strict, ob, (((1,), (0,)), ((), ())),
                                 preferred_element_type=jnp.float32) + carry
            carry = carry + jnp.sum(ob, axis=0, keepdims=True)
            p = jnp.sum(ob * (start + rb), axis=1, keepdims=True)
            pref[b * CB:(b + 1) * CB, :] = p.astype(jnp.int32)

    tril_incl = jnp.where(ei <= ej, 1.0, 0.0)
    nbs = lax.dot_general(nbf, tril_incl, (((1,), (0,)), ((), ())),
                          preferred_element_type=jnp.float32).astype(jnp.int32)
    nblk = nbs[0:1, E - 1:E]
    lane32 = lax.broadcasted_iota(jnp.int32, (1, 32), 1)
    bev = jnp.zeros((1, 32), jnp.int32)
    last_e = jnp.zeros((1, 1), jnp.int32)
    for e in range(E):
        nbs_e = nbs[0:1, e:e + 1]
        bev = bev + jnp.where(nbs_e <= lane32, 1, 0)
        last_e = last_e + jnp.where(nbs_e < nblk, 1, 0)
    val = jnp.where(lane32 < nblk, bev, last_e)
    val = jnp.where(lane32 == MAXB, nblk, val)
    bemeta_ref[...] = val


GCH = 32
NGC = 128 // GCH


def _gather_body(x_hbm, pos0_hbm, pos1_hbm, xs_hbm, idx_v, idxa_v, idxb_v,
                 rows_v, seml, sems):
    c = lax.axis_index("c")
    s = lax.axis_index("s")
    u = s * 2 + c
    tok0 = (u % 16) * 128

    pltpu.sync_copy(pos0_hbm.at[u % 16], idxa_v)
    pltpu.sync_copy(pos1_hbm.at[u % 16], idxb_v)
    msk = jnp.where(jnp.full((16,), u, jnp.int32) < 16, 1, 0)
    for ch in range(NGC):
        for v in range(GCH // 16):
            sl = pl.ds(v * 16, 16)
            idx_v[ch, sl] = (idxa_v[ch, sl] * msk
                             + idxb_v[ch, sl] * (1 - msk))

    def load(ch, buf):
        return pltpu.async_copy(x_hbm.at[pl.ds(tok0 + ch * GCH, GCH)],
                                rows_v.at[buf], seml)

    lcp = load(0, 0)
    scp = None
    for ch in range(NGC):
        buf = ch % 2
        lcp.wait()
        if scp is not None:
            scp.wait()
        if ch + 1 < NGC:
            lcp = load(ch + 1, 1 - buf)
        scp = pltpu.async_copy(rows_v.at[buf], xs_hbm.at[idx_v.at[ch]], sems)
    scp.wait()


def _ffn_body(be_sm, xs_ref, w1_ref, w3_ref, w2_ref, ys_ref):
    b = pl.program_id(0)
    nblk = be_sm[MAXB]

    @pl.when(b < nblk)
    def _():
        xb = xs_ref[...]
        a = lax.dot_general(xb, w1_ref[0], (((1,), (1,)), ((), ())),
                            preferred_element_type=jnp.float32)
        u = lax.dot_general(xb, w3_ref[0], (((1,), (1,)), ((), ())),
                            preferred_element_type=jnp.float32)
        h = (a * jax.nn.sigmoid(a)) * u
        y = lax.dot_general(h, w2_ref[0], (((1,), (1,)), ((), ())),
                            preferred_element_type=jnp.float32)
        ys_ref[...] = y


CCH = 16
NCH = 64 // CCH


def _combine_body(ys_hbm, pos0_hbm, pos1_hbm, ww0_hbm, ww1_hbm, out_hbm,
                  idx0_v, idx1_v, w0_v, w1_v, r0_v, r1_v, o_v, sem):
    c = lax.axis_index("c")
    s = lax.axis_index("s")
    u = s * 2 + c
    tb = u * 64

    pltpu.sync_copy(pos0_hbm.at[pl.ds(tb, 64)], idx0_v)
    pltpu.sync_copy(pos1_hbm.at[pl.ds(tb, 64)], idx1_v)
    pltpu.sync_copy(ww0_hbm.at[pl.ds(tb, 64)], w0_v)
    pltpu.sync_copy(ww1_hbm.at[pl.ds(tb, 64)], w1_v)

    def issue(ch, buf):
        a = pltpu.async_copy(ys_hbm.at[idx0_v.at[pl.ds(ch * CCH, CCH)]],
                             r0_v.at[buf], sem)
        b = pltpu.async_copy(ys_hbm.at[idx1_v.at[pl.ds(ch * CCH, CCH)]],
                             r1_v.at[buf], sem)
        return a, b

    cps = issue(0, 0)
    for ch in range(NCH):
        cps[0].wait()
        cps[1].wait()
        buf = ch % 2
        if ch + 1 < NCH:
            cps = issue(ch + 1, 1 - buf)

        def tok_step(t, _, ch=ch, buf=buf):
            s0 = w0_v[ch * CCH + t]
            s1 = w1_v[ch * CCH + t]
            for v in range(D // 16):
                sl = pl.ds(v * 16, 16)
                o_v[t, sl] = (r0_v[buf, t, sl] * s0
                              + r1_v[buf, t, sl] * s1)
            return 0

        lax.fori_loop(0, CCH, tok_step, 0)
        pltpu.sync_copy(o_v, out_hbm.at[pl.ds(tb + ch * CCH, CCH)])


_MESH = plsc.VectorSubcoreMesh(core_axis_name="c", subcore_axis_name="s")


@jax.jit
def _moe(hidden_states, gate_w, w1, w2, w3):
    x = hidden_states.reshape(T, D)

    logits, pos0, pos1, ww0, ww1, bemeta = pl.pallas_call(
        _router_body,
        out_shape=(
            jax.ShapeDtypeStruct((T, E), jnp.float32),
            jax.ShapeDtypeStruct((T, 1), jnp.int32),
            jax.ShapeDtypeStruct((T, 1), jnp.int32),
            jax.ShapeDtypeStruct((T, 16), jnp.float32),
            jax.ShapeDtypeStruct((T, 16), jnp.float32),
            jax.ShapeDtypeStruct((1, 32), jnp.int32),
        ),
    )(x, gate_w)

    p0g = pos0.reshape(16, NGC, GCH)
    p1g = pos1.reshape(16, NGC, GCH)
    xs = pl.kernel(
        _gather_body,
        mesh=_MESH,
        out_type=[jax.ShapeDtypeStruct((PADN, D), jnp.float32)],
        scratch_types=[
            pltpu.VMEM((NGC, GCH), jnp.int32),
            pltpu.VMEM((NGC, GCH), jnp.int32),
            pltpu.VMEM((NGC, GCH), jnp.int32),
            pltpu.VMEM((2, GCH, D), jnp.float32),
            pltpu.SemaphoreType.DMA,
            pltpu.SemaphoreType.DMA,
        ],
    )(x, p0g, p1g)[0]

    ys = pl.pallas_call(
        _ffn_body,
        grid_spec=pltpu.PrefetchScalarGridSpec(
            num_scalar_prefetch=1,
            grid=(MAXB,),
            in_specs=[
                pl.BlockSpec((BLK, D), lambda b, s: (b, 0)),
                pl.BlockSpec((1, F, D), lambda b, s: (s[b], 0, 0)),
                pl.BlockSpec((1, F, D), lambda b, s: (s[b], 0, 0)),
                pl.BlockSpec((1, D, F), lambda b, s: (s[b], 0, 0)),
            ],
            out_specs=pl.BlockSpec((BLK, D), lambda b, s: (b, 0)),
        ),
        out_shape=jax.ShapeDtypeStruct((PADN, D), jnp.float32),
        compiler_params=pltpu.CompilerParams(
            dimension_semantics=("arbitrary",),
        ),
    )(bemeta.reshape(32), xs, w1, w3, w2)

    final = pl.kernel(
        _combine_body,
        mesh=_MESH,
        out_type=[jax.ShapeDtypeStruct((T, D), jnp.float32)],
        scratch_types=[
            pltpu.VMEM((64,), jnp.int32),
            pltpu.VMEM((64,), jnp.int32),
            pltpu.VMEM((64, 16), jnp.float32),
            pltpu.VMEM((64, 16), jnp.float32),
            pltpu.VMEM((2, CCH, D), jnp.float32),
            pltpu.VMEM((2, CCH, D), jnp.float32),
            pltpu.VMEM((CCH, D), jnp.float32),
            pltpu.SemaphoreType.DMA,
        ],
    )(ys, pos0.reshape(T), pos1.reshape(T), ww0, ww1)[0]

    return final.reshape(1, T, D), logits


def kernel(hidden_states, gate_w, w1, w2, w3):
    return _moe(hidden_states, gate_w, w1, w2, w3)
